# Initial kernel scaffold; baseline (speedup 1.0000x reference)
import jax
import jax.numpy as jnp
from jax.experimental import pallas as pl


def kernel(x):
    raise NotImplementedError("write your pallas kernel here")


# speedup vs baseline: 24.5576x; 24.5576x over previous
import functools

import jax
import jax.numpy as jnp
from jax import lax
from jax.experimental import pallas as pl
from jax.experimental.pallas import tpu as pltpu
from jax.experimental.pallas import tpu_sc as plsc

_C = 128
_LANES = 16
_NC = 2
_NS = 16
_NW = _NC * _NS
_R = 256


def _build(n):
    rpw = n // _NW
    nchunk = rpw // _R
    tile_words = _R * _C

    mesh = plsc.VectorSubcoreMesh(core_axis_name="c", subcore_axis_name="s")

    @functools.partial(
        pl.kernel,
        out_type=jax.ShapeDtypeStruct((n * _C,), jnp.float32),
        mesh=mesh,
        compiler_params=pltpu.CompilerParams(needs_layout_passes=False),
        scratch_types=[
            pltpu.VMEM((rpw,), jnp.float32),
            pltpu.VMEM((2 * tile_words,), jnp.float32),
            pltpu.SemaphoreType.DMA,
            pltpu.SemaphoreType.DMA,
        ],
    )
    def run(x_hbm, out_hbm, lab_v, tiles, sem0, sem1):
        wid = lax.axis_index("s") * _NC + lax.axis_index("c")
        base = wid * rpw
        sems = (sem0, sem1)

        pltpu.sync_copy(x_hbm.at[pl.ds(base, rpw)], lab_v)

        zeros16 = jnp.zeros((_LANES,), jnp.float32)

        def zbody(i, _):
            tiles[pl.ds(i * _LANES, _LANES)] = zeros16
            return 0

        lax.fori_loop(0, (2 * tile_words) // _LANES, zbody, 0)

        iota = lax.iota(jnp.int32, _LANES)
        fives = jnp.full((_LANES,), 5.0, jnp.float32)

        def scatter_chunk(kk, b, val):
            def jbody(j, _):
                lv = lab_v[pl.ds(kk * _R + j * _LANES, _LANES)]
                col = jnp.clip(lv.astype(jnp.int32), 0, _C - 1)
                idx = (b * tile_words + j * (_LANES * _C)) + iota * _C + col
                plsc.store_scatter(tiles, [idx], val)
                return 0

            lax.fori_loop(0, _R // _LANES, jbody, 0)

        def dma(kk, b):
            src = tiles.at[pl.ds(b * tile_words, tile_words)]
            dst = out_hbm.at[pl.ds(base * _C + kk * tile_words, tile_words)]
            return pltpu.make_async_copy(src, dst, sems[b])

        for b in range(2):
            scatter_chunk(b, b, fives)
            dma(b, b).start()

        def loop_body(i, _):
            k0 = 2 * i
            for b in range(2):
                kk = k0 + b
                dma(kk - 2, b).wait()
                scatter_chunk(kk - 2, b, zeros16)
                scatter_chunk(kk, b, fives)
                dma(kk, b).start()
            return 0

        lax.fori_loop(1, nchunk // 2, loop_body, 0)

        for b in range(2):
            dma(nchunk - 2 + b, b).wait()

    return run


def kernel(x):
    n = x.shape[0]
    out_flat = _build(n)(x.reshape(-1))
    return out_flat.reshape(n, _C)


# speedup vs baseline: 26.7264x; 1.0883x over previous
import functools

import jax
import jax.numpy as jnp
from jax import lax
from jax.experimental import pallas as pl
from jax.experimental.pallas import tpu as pltpu
from jax.experimental.pallas import tpu_sc as plsc

_C = 128
_LANES = 16
_NC = 2
_NS = 16
_NW = _NC * _NS
_R = 256


def _build(n):
    rpw = n // _NW
    nchunk = rpw // _R
    tile_words = _R * _C

    mesh = plsc.VectorSubcoreMesh(core_axis_name="c", subcore_axis_name="s")

    @functools.partial(
        pl.kernel,
        out_type=jax.ShapeDtypeStruct((n * _C,), jnp.float32),
        mesh=mesh,
        compiler_params=pltpu.CompilerParams(needs_layout_passes=False),
        scratch_types=[
            pltpu.VMEM((rpw,), jnp.float32),
            pltpu.VMEM((2 * tile_words,), jnp.float32),
            pltpu.SemaphoreType.DMA,
            pltpu.SemaphoreType.DMA,
            pltpu.SemaphoreType.DMA,
        ],
    )
    def run(x_hbm, out_hbm, lab_v, tiles, sem0, sem1, sem_lab):
        wid = lax.axis_index("s") * _NC + lax.axis_index("c")
        base = wid * rpw
        sems = (sem0, sem1)

        lab_cp = pltpu.make_async_copy(x_hbm.at[pl.ds(base, rpw)], lab_v,
                                       sem_lab)
        lab_cp.start()

        zeros16 = jnp.zeros((_LANES,), jnp.float32)
        zunroll = 16

        def zbody(i, _):
            for u in range(zunroll):
                tiles[pl.ds((i * zunroll + u) * _LANES, _LANES)] = zeros16
            return 0

        lax.fori_loop(0, (2 * tile_words) // (_LANES * zunroll), zbody, 0)
        lab_cp.wait()

        iota = lax.iota(jnp.int32, _LANES)
        fives = jnp.full((_LANES,), 5.0, jnp.float32)

        def scatter_chunk(kk, b, val):
            def jbody(j, _):
                lv = lab_v[pl.ds(kk * _R + j * _LANES, _LANES)]
                col = jnp.clip(lv.astype(jnp.int32), 0, _C - 1)
                idx = (b * tile_words + j * (_LANES * _C)) + iota * _C + col
                plsc.store_scatter(tiles, [idx], val)
                return 0

            lax.fori_loop(0, _R // _LANES, jbody, 0)

        def dma(kk, b):
            src = tiles.at[pl.ds(b * tile_words, tile_words)]
            dst = out_hbm.at[pl.ds(base * _C + kk * tile_words, tile_words)]
            return pltpu.make_async_copy(src, dst, sems[b])

        for b in range(2):
            scatter_chunk(b, b, fives)
            dma(b, b).start()

        def loop_body(i, _):
            k0 = 2 * i
            for b in range(2):
                kk = k0 + b
                dma(kk - 2, b).wait()
                scatter_chunk(kk - 2, b, zeros16)
                scatter_chunk(kk, b, fives)
                dma(kk, b).start()
            return 0

        lax.fori_loop(1, nchunk // 2, loop_body, 0)

        for b in range(2):
            dma(nchunk - 2 + b, b).wait()

    return run


def kernel(x):
    n = x.shape[0]
    out_flat = _build(n)(x.reshape(-1))
    return out_flat.reshape(n, _C)
